# Initial kernel scaffold; baseline (speedup 1.0000x reference)
import jax
import jax.numpy as jnp
from jax.experimental import pallas as pl


def kernel(x, edge_index, W1, b1, W2, b2):
    raise NotImplementedError("write your pallas kernel here")


# speedup vs baseline: 14.0931x; 14.0931x over previous
import dataclasses
import functools

import jax
import jax.numpy as jnp
from jax import lax
from jax.experimental import pallas as pl
from jax.experimental.pallas import tpu as pltpu
from jax.experimental.pallas import tpu_sc as plsc

N = 10000
E = 320000
D = 128

NC = 2
NS = 16
NW = NC * NS
E_PER_W = E // NW
CH = 80
N_CH = E_PER_W // CH
R_MAIN = 632
R_LAST = N - (NS - 1) * R_MAIN
ZB = 8

BM = 1000

_mesh = plsc.VectorSubcoreMesh(core_axis_name="c", subcore_axis_name="s")


def _per_tile_rows(sid, fn):
    @pl.when(sid < NS - 1)
    def _():
        fn(sid * R_MAIN, R_MAIN)

    @pl.when(sid == NS - 1)
    def _():
        fn((NS - 1) * R_MAIN, R_LAST)


def _zero_fill(buf, rows, width):
    @pl.loop(0, rows)
    def _(i):
        @pl.loop(0, width // 16)
        def _(j):
            buf[i, pl.ds(j * 16, 16)] = jnp.zeros((16,), jnp.float32)


RD_MAIN = 640
RD_LAST = N - (NS - 1) * RD_MAIN
RD_LAST_PAD = 512
N_PAD = (NS - 1) * RD_MAIN + RD_LAST_PAD


def _sc_degree(dst):
    @functools.partial(
        pl.kernel,
        out_type=jax.ShapeDtypeStruct((NC, N, 16), jnp.float32),
        mesh=_mesh,
        scratch_types=[
            pltpu.VMEM((E_PER_W,), jnp.int32),
            pltpu.VMEM((N_PAD,), jnp.float32),
            pltpu.VMEM((NS, RD_MAIN), jnp.float32),
            pltpu.VMEM((RD_MAIN, 16), jnp.float32),
            pltpu.VMEM_SHARED((NS, N_PAD), jnp.float32),
        ],
        compiler_params=dataclasses.replace(
            pltpu.CompilerParams(), needs_layout_passes=False
        ),
    )
    def k(dst_hbm, out_hbm, didx, hist, lbuf, obuf, stage):
        cid = lax.axis_index("c")
        sid = lax.axis_index("s")
        wid = sid * NC + cid

        @pl.loop(0, N_PAD // 16)
        def _(i):
            hist[pl.ds(i * 16, 16)] = jnp.zeros((16,), jnp.float32)

        pltpu.sync_copy(dst_hbm.at[pl.ds(wid * E_PER_W, E_PER_W)], didx)
        ones16 = jnp.ones((16,), jnp.float32)

        @pl.loop(0, E_PER_W // 16)
        def _(i):
            dvec = didx[pl.ds(i * 16, 16)]
            plsc.addupdate_scatter(hist, [dvec], ones16)

        pltpu.sync_copy(hist, stage.at[sid])
        plsc.subcore_barrier()

        def _reduce(start, rsize, wsize):
            pltpu.sync_copy(stage.at[:, pl.ds(start, rsize)],
                            lbuf.at[:, pl.ds(0, rsize)])
            lane0 = jnp.zeros((16,), jnp.int32)
            rows0 = lax.iota(jnp.int32, 16)

            @pl.loop(0, rsize // 16)
            def _(i):
                s = lbuf[0, pl.ds(i * 16, 16)]
                for j in range(1, NS):
                    s = s + lbuf[j, pl.ds(i * 16, 16)]
                plsc.store_scatter(obuf, [rows0 + i * 16, lane0], s)

            pltpu.sync_copy(obuf.at[pl.ds(0, wsize)],
                            out_hbm.at[cid, pl.ds(start, wsize)])

        @pl.when(sid < NS - 1)
        def _():
            _reduce(sid * RD_MAIN, RD_MAIN, RD_MAIN)

        @pl.when(sid == NS - 1)
        def _():
            _reduce((NS - 1) * RD_MAIN, RD_LAST_PAD, RD_LAST)

    return k(dst)


def _sc_scatter(z, src, dst):
    @functools.partial(
        pl.kernel,
        out_type=jax.ShapeDtypeStruct((NC, N, D), jnp.float32),
        mesh=_mesh,
        scratch_types=[
            pltpu.VMEM((CH,), jnp.int32),
            pltpu.VMEM((CH,), jnp.int32),
            pltpu.VMEM((CH, D), jnp.float32),
            pltpu.VMEM((ZB, D), jnp.float32),
            pltpu.VMEM_SHARED((N, D), jnp.float32),
            pltpu.SemaphoreType.DMA,
        ],
    )
    def k(z_hbm, src_hbm, dst_hbm, out_hbm, sidx, didx, rows, zbuf, acc, sem):
        cid = lax.axis_index("c")
        sid = lax.axis_index("s")
        wid = sid * NC + cid

        _zero_fill(zbuf, ZB, D)

        def _init(start, size):
            @pl.loop(0, size // ZB)
            def _(i):
                pltpu.sync_copy(zbuf, acc.at[pl.ds(start + i * ZB, ZB)])

        _per_tile_rows(sid, _init)
        plsc.subcore_barrier()

        base = wid * E_PER_W
        @pl.loop(0, N_CH)
        def _(j):
            off = base + j * CH
            pltpu.sync_copy(src_hbm.at[pl.ds(off, CH)], sidx)
            pltpu.sync_copy(dst_hbm.at[pl.ds(off, CH)], didx)
            pltpu.async_copy(z_hbm.at[sidx], rows, sem).wait()
            pltpu.sync_copy(rows, acc.at[didx], add=True)

        plsc.subcore_barrier()

        def _drain(start, size):
            pltpu.sync_copy(
                acc.at[pl.ds(start, size)],
                out_hbm.at[cid, pl.ds(start, size)],
            )

        _per_tile_rows(sid, _drain)

    return k(z, src, dst)


def _dinv_block(degp):
    return lax.rsqrt(degp[0, :, 0:1] + degp[1, :, 0:1] + 1.0)


def _mm_body(x_ref, w_ref, o_ref):
    o_ref[...] = jnp.dot(
        x_ref[...], w_ref[...],
        preferred_element_type=jnp.float32, precision=lax.Precision.HIGHEST,
    )


def _tc_matmul(x, W):
    return pl.pallas_call(
        _mm_body,
        grid=(N // BM,),
        in_specs=[
            pl.BlockSpec((BM, D), lambda i: (i, 0)),
            pl.BlockSpec((D, D), lambda i: (0, 0)),
        ],
        out_specs=pl.BlockSpec((BM, D), lambda i: (i, 0)),
        out_shape=jax.ShapeDtypeStruct((N, D), jnp.float32),
    )(x, W)


def _z_body(degp_ref, y_ref, z_ref):
    z_ref[...] = _dinv_block(degp_ref) * y_ref[...]


def _tc_scale(degp, y):
    return pl.pallas_call(
        _z_body,
        grid=(N // BM,),
        in_specs=[
            pl.BlockSpec((NC, BM, 16), lambda i: (0, i, 0)),
            pl.BlockSpec((BM, D), lambda i: (i, 0)),
        ],
        out_specs=pl.BlockSpec((BM, D), lambda i: (i, 0)),
        out_shape=jax.ShapeDtypeStruct((N, D), jnp.float32),
    )(degp, y)


def _mid_body(degp_ref, p_ref, z1_ref, b1_ref, w2_ref, z2_ref):
    dinv = _dinv_block(degp_ref)
    agg = p_ref[0] + p_ref[1] + z1_ref[...]
    h = jnp.maximum(dinv * agg + b1_ref[...], 0.0)
    y2 = jnp.dot(
        h, w2_ref[...],
        preferred_element_type=jnp.float32, precision=lax.Precision.HIGHEST,
    )
    z2_ref[...] = dinv * y2


def _tc_mid(degp, p, z1, b1, W2):
    return pl.pallas_call(
        _mid_body,
        grid=(N // BM,),
        in_specs=[
            pl.BlockSpec((NC, BM, 16), lambda i: (0, i, 0)),
            pl.BlockSpec((NC, BM, D), lambda i: (0, i, 0)),
            pl.BlockSpec((BM, D), lambda i: (i, 0)),
            pl.BlockSpec((1, D), lambda i: (0, 0)),
            pl.BlockSpec((D, D), lambda i: (0, 0)),
        ],
        out_specs=pl.BlockSpec((BM, D), lambda i: (i, 0)),
        out_shape=jax.ShapeDtypeStruct((N, D), jnp.float32),
    )(degp, p, z1, b1, W2)


def _final_body(degp_ref, q_ref, z2_ref, b2_ref, o_ref):
    dinv = _dinv_block(degp_ref)
    g = dinv * (q_ref[0] + q_ref[1] + z2_ref[...]) + b2_ref[...]
    m = jnp.max(g, axis=1, keepdims=True)
    lse = m + jnp.log(jnp.sum(jnp.exp(g - m), axis=1, keepdims=True))
    o_ref[...] = g - lse


def _tc_final(degp, q, z2, b2):
    return pl.pallas_call(
        _final_body,
        grid=(N // BM,),
        in_specs=[
            pl.BlockSpec((NC, BM, 16), lambda i: (0, i, 0)),
            pl.BlockSpec((NC, BM, D), lambda i: (0, i, 0)),
            pl.BlockSpec((BM, D), lambda i: (i, 0)),
            pl.BlockSpec((1, D), lambda i: (0, 0)),
        ],
        out_specs=pl.BlockSpec((BM, D), lambda i: (i, 0)),
        out_shape=jax.ShapeDtypeStruct((N, D), jnp.float32),
    )(degp, q, z2, b2)


def kernel(x, edge_index, W1, b1, W2, b2):
    ei = edge_index.astype(jnp.int32)
    src = ei[0]
    dst = ei[1]
    b1 = b1.reshape(1, D)
    b2 = b2.reshape(1, D)

    degp = _sc_degree(dst)
    y1 = _tc_matmul(x, W1)
    z1 = _tc_scale(degp, y1)
    p = _sc_scatter(z1, src, dst)
    z2 = _tc_mid(degp, p, z1, b1, W2)
    q = _sc_scatter(z2, src, dst)
    return _tc_final(degp, q, z2, b2)


# speedup vs baseline: 21.7632x; 1.5442x over previous
import dataclasses
import functools

import jax
import jax.numpy as jnp
from jax import lax
from jax.experimental import pallas as pl
from jax.experimental.pallas import tpu as pltpu
from jax.experimental.pallas import tpu_sc as plsc

N = 10000
E = 320000
D = 128

NC = 2
NS = 16
NW = NC * NS
E_PER_W = E // NW
CH = 80
N_CH = E_PER_W // CH
R_MAIN = 632
R_LAST = N - (NS - 1) * R_MAIN
ZB = 8

BM = 1000

_mesh = plsc.VectorSubcoreMesh(core_axis_name="c", subcore_axis_name="s")


def _per_tile_rows(sid, fn):
    @pl.when(sid < NS - 1)
    def _():
        fn(sid * R_MAIN, R_MAIN)

    @pl.when(sid == NS - 1)
    def _():
        fn((NS - 1) * R_MAIN, R_LAST)


def _zero_fill(buf, rows, width):
    @pl.loop(0, rows)
    def _(i):
        @pl.loop(0, width // 16)
        def _(j):
            buf[i, pl.ds(j * 16, 16)] = jnp.zeros((16,), jnp.float32)


RD_MAIN = 640
RD_LAST = N - (NS - 1) * RD_MAIN
RD_LAST_PAD = 512
N_PAD = (NS - 1) * RD_MAIN + RD_LAST_PAD


def _sc_degree(dst):
    @functools.partial(
        pl.kernel,
        out_type=jax.ShapeDtypeStruct((NC, N, 16), jnp.float32),
        mesh=_mesh,
        scratch_types=[
            pltpu.VMEM((E_PER_W,), jnp.int32),
            pltpu.VMEM((N_PAD,), jnp.float32),
            pltpu.VMEM((NS, RD_MAIN), jnp.float32),
            pltpu.VMEM((RD_MAIN, 16), jnp.float32),
            pltpu.VMEM_SHARED((NS, N_PAD), jnp.float32),
        ],
        compiler_params=dataclasses.replace(
            pltpu.CompilerParams(), needs_layout_passes=False
        ),
    )
    def k(dst_hbm, out_hbm, didx, hist, lbuf, obuf, stage):
        cid = lax.axis_index("c")
        sid = lax.axis_index("s")
        wid = sid * NC + cid

        @pl.loop(0, N_PAD // 16)
        def _(i):
            hist[pl.ds(i * 16, 16)] = jnp.zeros((16,), jnp.float32)

        pltpu.sync_copy(dst_hbm.at[pl.ds(wid * E_PER_W, E_PER_W)], didx)
        ones16 = jnp.ones((16,), jnp.float32)

        @pl.loop(0, E_PER_W // 16)
        def _(i):
            dvec = didx[pl.ds(i * 16, 16)]
            plsc.addupdate_scatter(hist, [dvec], ones16)

        pltpu.sync_copy(hist, stage.at[sid])
        plsc.subcore_barrier()

        def _reduce(start, rsize, wsize):
            pltpu.sync_copy(stage.at[:, pl.ds(start, rsize)],
                            lbuf.at[:, pl.ds(0, rsize)])
            lane0 = jnp.zeros((16,), jnp.int32)
            rows0 = lax.iota(jnp.int32, 16)

            @pl.loop(0, rsize // 16)
            def _(i):
                s = lbuf[0, pl.ds(i * 16, 16)]
                for j in range(1, NS):
                    s = s + lbuf[j, pl.ds(i * 16, 16)]
                plsc.store_scatter(obuf, [rows0 + i * 16, lane0], s)

            pltpu.sync_copy(obuf.at[pl.ds(0, wsize)],
                            out_hbm.at[cid, pl.ds(start, wsize)])

        @pl.when(sid < NS - 1)
        def _():
            _reduce(sid * RD_MAIN, RD_MAIN, RD_MAIN)

        @pl.when(sid == NS - 1)
        def _():
            _reduce((NS - 1) * RD_MAIN, RD_LAST_PAD, RD_LAST)

    return k(dst)


def _sc_scatter(z, src, dst):
    @functools.partial(
        pl.kernel,
        out_type=jax.ShapeDtypeStruct((NC, N, D), jnp.float32),
        mesh=_mesh,
        scratch_types=[
            pltpu.VMEM((2, CH), jnp.int32),
            pltpu.VMEM((2, CH), jnp.int32),
            pltpu.VMEM((CH, D), jnp.float32),
            pltpu.VMEM((CH, D), jnp.float32),
            pltpu.VMEM((ZB, D), jnp.float32),
            pltpu.VMEM_SHARED((N, D), jnp.float32),
            pltpu.SemaphoreType.DMA,
            pltpu.SemaphoreType.DMA,
            pltpu.SemaphoreType.DMA,
            pltpu.SemaphoreType.DMA,
        ],
    )
    def k(z_hbm, src_hbm, dst_hbm, out_hbm, sidx, didx, r0, r1, zbuf, acc,
          sg0, sg1, ss0, ss1):
        cid = lax.axis_index("c")
        sid = lax.axis_index("s")
        wid = sid * NC + cid
        base = wid * E_PER_W

        def _load_idx(j, b):
            pltpu.sync_copy(src_hbm.at[pl.ds(base + j * CH, CH)], sidx.at[b])
            pltpu.sync_copy(dst_hbm.at[pl.ds(base + j * CH, CH)], didx.at[b])

        _zero_fill(zbuf, ZB, D)

        def _init(start, size):
            @pl.loop(0, size // ZB)
            def _(i):
                pltpu.sync_copy(zbuf, acc.at[pl.ds(start + i * ZB, ZB)])

        _per_tile_rows(sid, _init)
        plsc.subcore_barrier()

        _load_idx(0, 0)
        _load_idx(1, 1)
        pltpu.async_copy(z_hbm.at[sidx.at[0]], r0, sg0)
        pltpu.async_copy(z_hbm.at[sidx.at[1]], r1, sg1)

        @pl.loop(0, (N_CH - 1) // 2)
        def _(i):
            a = 2 * i
            pltpu.make_async_copy(z_hbm.at[sidx.at[0]], r0, sg0).wait()
            pltpu.async_copy(r0, acc.at[didx.at[0]], ss0, add=True)
            pltpu.make_async_copy(r0, acc.at[didx.at[0]], ss0).wait()

            @pl.when(a + 2 < N_CH)
            def _():
                _load_idx(a + 2, 0)
                pltpu.async_copy(z_hbm.at[sidx.at[0]], r0, sg0)

            pltpu.make_async_copy(z_hbm.at[sidx.at[1]], r1, sg1).wait()
            pltpu.async_copy(r1, acc.at[didx.at[1]], ss1, add=True)
            pltpu.make_async_copy(r1, acc.at[didx.at[1]], ss1).wait()

            @pl.when(a + 3 < N_CH)
            def _():
                _load_idx(a + 3, 1)
                pltpu.async_copy(z_hbm.at[sidx.at[1]], r1, sg1)

        pltpu.make_async_copy(z_hbm.at[sidx.at[0]], r0, sg0).wait()
        pltpu.sync_copy(r0, acc.at[didx.at[0]], add=True)

        plsc.subcore_barrier()

        def _drain(start, size):
            pltpu.sync_copy(
                acc.at[pl.ds(start, size)],
                out_hbm.at[cid, pl.ds(start, size)],
            )

        _per_tile_rows(sid, _drain)

    return k(z, src, dst)


def _dinv_block(degp):
    return lax.rsqrt(degp[0, :, 0:1] + degp[1, :, 0:1] + 1.0)


def _mm_body(x_ref, w_ref, o_ref):
    o_ref[...] = jnp.dot(
        x_ref[...], w_ref[...],
        preferred_element_type=jnp.float32, precision=lax.Precision.HIGHEST,
    )


def _tc_matmul(x, W):
    return pl.pallas_call(
        _mm_body,
        grid=(N // BM,),
        in_specs=[
            pl.BlockSpec((BM, D), lambda i: (i, 0)),
            pl.BlockSpec((D, D), lambda i: (0, 0)),
        ],
        out_specs=pl.BlockSpec((BM, D), lambda i: (i, 0)),
        out_shape=jax.ShapeDtypeStruct((N, D), jnp.float32),
    )(x, W)


def _z_body(degp_ref, y_ref, z_ref):
    z_ref[...] = _dinv_block(degp_ref) * y_ref[...]


def _tc_scale(degp, y):
    return pl.pallas_call(
        _z_body,
        grid=(N // BM,),
        in_specs=[
            pl.BlockSpec((NC, BM, 16), lambda i: (0, i, 0)),
            pl.BlockSpec((BM, D), lambda i: (i, 0)),
        ],
        out_specs=pl.BlockSpec((BM, D), lambda i: (i, 0)),
        out_shape=jax.ShapeDtypeStruct((N, D), jnp.float32),
    )(degp, y)


def _mid_body(degp_ref, p_ref, z1_ref, b1_ref, w2_ref, z2_ref):
    dinv = _dinv_block(degp_ref)
    agg = p_ref[0] + p_ref[1] + z1_ref[...]
    h = jnp.maximum(dinv * agg + b1_ref[...], 0.0)
    y2 = jnp.dot(
        h, w2_ref[...],
        preferred_element_type=jnp.float32, precision=lax.Precision.HIGHEST,
    )
    z2_ref[...] = dinv * y2


def _tc_mid(degp, p, z1, b1, W2):
    return pl.pallas_call(
        _mid_body,
        grid=(N // BM,),
        in_specs=[
            pl.BlockSpec((NC, BM, 16), lambda i: (0, i, 0)),
            pl.BlockSpec((NC, BM, D), lambda i: (0, i, 0)),
            pl.BlockSpec((BM, D), lambda i: (i, 0)),
            pl.BlockSpec((1, D), lambda i: (0, 0)),
            pl.BlockSpec((D, D), lambda i: (0, 0)),
        ],
        out_specs=pl.BlockSpec((BM, D), lambda i: (i, 0)),
        out_shape=jax.ShapeDtypeStruct((N, D), jnp.float32),
    )(degp, p, z1, b1, W2)


def _final_body(degp_ref, q_ref, z2_ref, b2_ref, o_ref):
    dinv = _dinv_block(degp_ref)
    g = dinv * (q_ref[0] + q_ref[1] + z2_ref[...]) + b2_ref[...]
    m = jnp.max(g, axis=1, keepdims=True)
    lse = m + jnp.log(jnp.sum(jnp.exp(g - m), axis=1, keepdims=True))
    o_ref[...] = g - lse


def _tc_final(degp, q, z2, b2):
    return pl.pallas_call(
        _final_body,
        grid=(N // BM,),
        in_specs=[
            pl.BlockSpec((NC, BM, 16), lambda i: (0, i, 0)),
            pl.BlockSpec((NC, BM, D), lambda i: (0, i, 0)),
            pl.BlockSpec((BM, D), lambda i: (i, 0)),
            pl.BlockSpec((1, D), lambda i: (0, 0)),
        ],
        out_specs=pl.BlockSpec((BM, D), lambda i: (i, 0)),
        out_shape=jax.ShapeDtypeStruct((N, D), jnp.float32),
    )(degp, q, z2, b2)


def kernel(x, edge_index, W1, b1, W2, b2):
    ei = edge_index.astype(jnp.int32)
    src = ei[0]
    dst = ei[1]
    b1 = b1.reshape(1, D)
    b2 = b2.reshape(1, D)

    degp = _sc_degree(dst)
    y1 = _tc_matmul(x, W1)
    z1 = _tc_scale(degp, y1)
    p = _sc_scatter(z1, src, dst)
    z2 = _tc_mid(degp, p, z1, b1, W2)
    q = _sc_scatter(z2, src, dst)
    return _tc_final(degp, q, z2, b2)


# speedup vs baseline: 21.9733x; 1.0097x over previous
import dataclasses
import functools

import jax
import jax.numpy as jnp
from jax import lax
from jax.experimental import pallas as pl
from jax.experimental.pallas import tpu as pltpu
from jax.experimental.pallas import tpu_sc as plsc

N = 10000
E = 320000
D = 128

NC = 2
NS = 16
NW = NC * NS
E_PER_W = E // NW
CH = 80
N_CH = E_PER_W // CH
R_MAIN = 632
R_LAST = N - (NS - 1) * R_MAIN
ZB = 8
NBUF = 4

BM = 1000

_mesh = plsc.VectorSubcoreMesh(core_axis_name="c", subcore_axis_name="s")


def _per_tile_rows(sid, fn):
    @pl.when(sid < NS - 1)
    def _():
        fn(sid * R_MAIN, R_MAIN)

    @pl.when(sid == NS - 1)
    def _():
        fn((NS - 1) * R_MAIN, R_LAST)


def _zero_fill(buf, rows, width):
    @pl.loop(0, rows)
    def _(i):
        @pl.loop(0, width // 16)
        def _(j):
            buf[i, pl.ds(j * 16, 16)] = jnp.zeros((16,), jnp.float32)


RD_MAIN = 640
RD_LAST = N - (NS - 1) * RD_MAIN
RD_LAST_PAD = 512
N_PAD = (NS - 1) * RD_MAIN + RD_LAST_PAD


def _sc_degree(dst):
    @functools.partial(
        pl.kernel,
        out_type=jax.ShapeDtypeStruct((NC, N, 16), jnp.float32),
        mesh=_mesh,
        scratch_types=[
            pltpu.VMEM((E_PER_W,), jnp.int32),
            pltpu.VMEM((N_PAD,), jnp.float32),
            pltpu.VMEM((NS, RD_MAIN), jnp.float32),
            pltpu.VMEM((RD_MAIN, 16), jnp.float32),
            pltpu.VMEM_SHARED((NS, N_PAD), jnp.float32),
        ],
        compiler_params=dataclasses.replace(
            pltpu.CompilerParams(), needs_layout_passes=False
        ),
    )
    def k(dst_hbm, out_hbm, didx, hist, lbuf, obuf, stage):
        cid = lax.axis_index("c")
        sid = lax.axis_index("s")
        wid = sid * NC + cid

        @pl.loop(0, N_PAD // 16)
        def _(i):
            hist[pl.ds(i * 16, 16)] = jnp.zeros((16,), jnp.float32)

        pltpu.sync_copy(dst_hbm.at[pl.ds(wid * E_PER_W, E_PER_W)], didx)
        ones16 = jnp.ones((16,), jnp.float32)

        @pl.loop(0, E_PER_W // 16)
        def _(i):
            dvec = didx[pl.ds(i * 16, 16)]
            plsc.addupdate_scatter(hist, [dvec], ones16)

        pltpu.sync_copy(hist, stage.at[sid])
        plsc.subcore_barrier()

        def _reduce(start, rsize, wsize):
            pltpu.sync_copy(stage.at[:, pl.ds(start, rsize)],
                            lbuf.at[:, pl.ds(0, rsize)])
            lane0 = jnp.zeros((16,), jnp.int32)
            rows0 = lax.iota(jnp.int32, 16)

            @pl.loop(0, rsize // 16)
            def _(i):
                s = lbuf[0, pl.ds(i * 16, 16)]
                for j in range(1, NS):
                    s = s + lbuf[j, pl.ds(i * 16, 16)]
                plsc.store_scatter(obuf, [rows0 + i * 16, lane0], s)

            pltpu.sync_copy(obuf.at[pl.ds(0, wsize)],
                            out_hbm.at[cid, pl.ds(start, wsize)])

        @pl.when(sid < NS - 1)
        def _():
            _reduce(sid * RD_MAIN, RD_MAIN, RD_MAIN)

        @pl.when(sid == NS - 1)
        def _():
            _reduce((NS - 1) * RD_MAIN, RD_LAST_PAD, RD_LAST)

    return k(dst)


def _sc_scatter(z, src, dst):
    @functools.partial(
        pl.kernel,
        out_type=jax.ShapeDtypeStruct((NC, N, D), jnp.float32),
        mesh=_mesh,
        scratch_types=[
            pltpu.VMEM((NBUF, CH), jnp.int32),
            pltpu.VMEM((NBUF, CH), jnp.int32),
        ]
        + [pltpu.VMEM((CH, D), jnp.float32)] * NBUF
        + [pltpu.VMEM((ZB, D), jnp.float32),
           pltpu.VMEM_SHARED((N, D), jnp.float32)]
        + [pltpu.SemaphoreType.DMA] * NBUF
        + [pltpu.SemaphoreType.DMA] * NBUF,
    )
    def k(z_hbm, src_hbm, dst_hbm, out_hbm, sidx, didx, *rest):
        r = rest[:NBUF]
        zbuf, acc = rest[NBUF], rest[NBUF + 1]
        sg = rest[NBUF + 2:2 * NBUF + 2]
        ss = rest[2 * NBUF + 2:]
        cid = lax.axis_index("c")
        sid = lax.axis_index("s")
        wid = sid * NC + cid
        base = wid * E_PER_W

        def _load_idx(j, b):
            pltpu.sync_copy(src_hbm.at[pl.ds(base + j * CH, CH)], sidx.at[b])
            pltpu.sync_copy(dst_hbm.at[pl.ds(base + j * CH, CH)], didx.at[b])

        _zero_fill(zbuf, ZB, D)

        def _init(start, size):
            @pl.loop(0, size // ZB)
            def _(i):
                pltpu.sync_copy(zbuf, acc.at[pl.ds(start + i * ZB, ZB)])

        _per_tile_rows(sid, _init)
        plsc.subcore_barrier()

        for b in range(NBUF):
            _load_idx(b, b)
            pltpu.async_copy(z_hbm.at[sidx.at[b]], r[b], sg[b])

        @pl.loop(0, (N_CH - 1) // NBUF)
        def _(i):
            a = i * NBUF
            for b in range(NBUF):
                pltpu.make_async_copy(z_hbm.at[sidx.at[b]], r[b], sg[b]).wait()
                pltpu.async_copy(r[b], acc.at[didx.at[b]], ss[b], add=True)
                pltpu.make_async_copy(r[b], acc.at[didx.at[b]], ss[b]).wait()

                @pl.when(a + b + NBUF < N_CH)
                def _():
                    _load_idx(a + b + NBUF, b)
                    pltpu.async_copy(z_hbm.at[sidx.at[b]], r[b], sg[b])

        rem = N_CH - NBUF * ((N_CH - 1) // NBUF)
        for b in range(rem):
            pltpu.make_async_copy(z_hbm.at[sidx.at[b]], r[b], sg[b]).wait()
            pltpu.sync_copy(r[b], acc.at[didx.at[b]], add=True)

        plsc.subcore_barrier()

        def _drain(start, size):
            pltpu.sync_copy(
                acc.at[pl.ds(start, size)],
                out_hbm.at[cid, pl.ds(start, size)],
            )

        _per_tile_rows(sid, _drain)

    return k(z, src, dst)


def _dinv_block(degp):
    return lax.rsqrt(degp[0, :, 0:1] + degp[1, :, 0:1] + 1.0)


def _mm_body(x_ref, w_ref, o_ref):
    o_ref[...] = jnp.dot(
        x_ref[...], w_ref[...],
        preferred_element_type=jnp.float32, precision=lax.Precision.HIGHEST,
    )


def _tc_matmul(x, W):
    return pl.pallas_call(
        _mm_body,
        grid=(N // BM,),
        in_specs=[
            pl.BlockSpec((BM, D), lambda i: (i, 0)),
            pl.BlockSpec((D, D), lambda i: (0, 0)),
        ],
        out_specs=pl.BlockSpec((BM, D), lambda i: (i, 0)),
        out_shape=jax.ShapeDtypeStruct((N, D), jnp.float32),
    )(x, W)


def _z_body(degp_ref, y_ref, z_ref):
    z_ref[...] = _dinv_block(degp_ref) * y_ref[...]


def _tc_scale(degp, y):
    return pl.pallas_call(
        _z_body,
        grid=(N // BM,),
        in_specs=[
            pl.BlockSpec((NC, BM, 16), lambda i: (0, i, 0)),
            pl.BlockSpec((BM, D), lambda i: (i, 0)),
        ],
        out_specs=pl.BlockSpec((BM, D), lambda i: (i, 0)),
        out_shape=jax.ShapeDtypeStruct((N, D), jnp.float32),
    )(degp, y)


def _mid_body(degp_ref, p_ref, z1_ref, b1_ref, w2_ref, z2_ref):
    dinv = _dinv_block(degp_ref)
    agg = p_ref[0] + p_ref[1] + z1_ref[...]
    h = jnp.maximum(dinv * agg + b1_ref[...], 0.0)
    y2 = jnp.dot(
        h, w2_ref[...],
        preferred_element_type=jnp.float32, precision=lax.Precision.HIGHEST,
    )
    z2_ref[...] = dinv * y2


def _tc_mid(degp, p, z1, b1, W2):
    return pl.pallas_call(
        _mid_body,
        grid=(N // BM,),
        in_specs=[
            pl.BlockSpec((NC, BM, 16), lambda i: (0, i, 0)),
            pl.BlockSpec((NC, BM, D), lambda i: (0, i, 0)),
            pl.BlockSpec((BM, D), lambda i: (i, 0)),
            pl.BlockSpec((1, D), lambda i: (0, 0)),
            pl.BlockSpec((D, D), lambda i: (0, 0)),
        ],
        out_specs=pl.BlockSpec((BM, D), lambda i: (i, 0)),
        out_shape=jax.ShapeDtypeStruct((N, D), jnp.float32),
    )(degp, p, z1, b1, W2)


def _final_body(degp_ref, q_ref, z2_ref, b2_ref, o_ref):
    dinv = _dinv_block(degp_ref)
    g = dinv * (q_ref[0] + q_ref[1] + z2_ref[...]) + b2_ref[...]
    m = jnp.max(g, axis=1, keepdims=True)
    lse = m + jnp.log(jnp.sum(jnp.exp(g - m), axis=1, keepdims=True))
    o_ref[...] = g - lse


def _tc_final(degp, q, z2, b2):
    return pl.pallas_call(
        _final_body,
        grid=(N // BM,),
        in_specs=[
            pl.BlockSpec((NC, BM, 16), lambda i: (0, i, 0)),
            pl.BlockSpec((NC, BM, D), lambda i: (0, i, 0)),
            pl.BlockSpec((BM, D), lambda i: (i, 0)),
            pl.BlockSpec((1, D), lambda i: (0, 0)),
        ],
        out_specs=pl.BlockSpec((BM, D), lambda i: (i, 0)),
        out_shape=jax.ShapeDtypeStruct((N, D), jnp.float32),
    )(degp, q, z2, b2)


def kernel(x, edge_index, W1, b1, W2, b2):
    ei = edge_index.astype(jnp.int32)
    src = ei[0]
    dst = ei[1]
    b1 = b1.reshape(1, D)
    b2 = b2.reshape(1, D)

    degp = _sc_degree(dst)
    y1 = _tc_matmul(x, W1)
    z1 = _tc_scale(degp, y1)
    p = _sc_scatter(z1, src, dst)
    z2 = _tc_mid(degp, p, z1, b1, W2)
    q = _sc_scatter(z2, src, dst)
    return _tc_final(degp, q, z2, b2)


# speedup vs baseline: 35.1603x; 1.6001x over previous
import dataclasses
import functools

import jax
import jax.numpy as jnp
from jax import lax
from jax.experimental import pallas as pl
from jax.experimental.pallas import tpu as pltpu
from jax.experimental.pallas import tpu_sc as plsc

N = 10000
E = 320000
D = 128

NC = 2
NS = 16
NW = NC * NS
E_PER_W = E // NW
CH = 80
N_CH = E_PER_W // CH
R_MAIN = 632
R_LAST = N - (NS - 1) * R_MAIN
ZB = 8
NBUF = 4
RING = 2 * NBUF

BM = 1000

_mesh = plsc.VectorSubcoreMesh(core_axis_name="c", subcore_axis_name="s")


def _per_tile_rows(sid, fn):
    @pl.when(sid < NS - 1)
    def _():
        fn(sid * R_MAIN, R_MAIN)

    @pl.when(sid == NS - 1)
    def _():
        fn((NS - 1) * R_MAIN, R_LAST)


def _zero_fill(buf, rows, width):
    @pl.loop(0, rows)
    def _(i):
        @pl.loop(0, width // 16)
        def _(j):
            buf[i, pl.ds(j * 16, 16)] = jnp.zeros((16,), jnp.float32)


RD_MAIN = 640
RD_LAST = N - (NS - 1) * RD_MAIN
RD_LAST_PAD = 512
N_PAD = (NS - 1) * RD_MAIN + RD_LAST_PAD


def _sc_degree(dst):
    @functools.partial(
        pl.kernel,
        out_type=jax.ShapeDtypeStruct((NC, N, 16), jnp.float32),
        mesh=_mesh,
        scratch_types=[
            pltpu.VMEM((E_PER_W,), jnp.int32),
            pltpu.VMEM((N_PAD,), jnp.float32),
            pltpu.VMEM((NS, RD_MAIN), jnp.float32),
            pltpu.VMEM((RD_MAIN, 16), jnp.float32),
            pltpu.VMEM_SHARED((NS, N_PAD), jnp.float32),
        ],
        compiler_params=dataclasses.replace(
            pltpu.CompilerParams(), needs_layout_passes=False
        ),
    )
    def k(dst_hbm, out_hbm, didx, hist, lbuf, obuf, stage):
        cid = lax.axis_index("c")
        sid = lax.axis_index("s")
        wid = sid * NC + cid

        @pl.loop(0, N_PAD // 16)
        def _(i):
            hist[pl.ds(i * 16, 16)] = jnp.zeros((16,), jnp.float32)

        pltpu.sync_copy(dst_hbm.at[pl.ds(wid * E_PER_W, E_PER_W)], didx)
        ones16 = jnp.ones((16,), jnp.float32)

        @pl.loop(0, E_PER_W // 16)
        def _(i):
            dvec = didx[pl.ds(i * 16, 16)]
            plsc.addupdate_scatter(hist, [dvec], ones16)

        pltpu.sync_copy(hist, stage.at[sid])
        plsc.subcore_barrier()

        def _reduce(start, rsize, wsize):
            pltpu.sync_copy(stage.at[:, pl.ds(start, rsize)],
                            lbuf.at[:, pl.ds(0, rsize)])
            lane0 = jnp.zeros((16,), jnp.int32)
            rows0 = lax.iota(jnp.int32, 16)

            @pl.loop(0, rsize // 16)
            def _(i):
                s = lbuf[0, pl.ds(i * 16, 16)]
                for j in range(1, NS):
                    s = s + lbuf[j, pl.ds(i * 16, 16)]
                plsc.store_scatter(obuf, [rows0 + i * 16, lane0], s)

            pltpu.sync_copy(obuf.at[pl.ds(0, wsize)],
                            out_hbm.at[cid, pl.ds(start, wsize)])

        @pl.when(sid < NS - 1)
        def _():
            _reduce(sid * RD_MAIN, RD_MAIN, RD_MAIN)

        @pl.when(sid == NS - 1)
        def _():
            _reduce((NS - 1) * RD_MAIN, RD_LAST_PAD, RD_LAST)

    return k(dst)


def _maybe(cond, fn):
    if isinstance(cond, bool):
        if cond:
            fn()
    else:
        pl.when(cond)(fn)


def _sc_scatter(z, sd):
    @functools.partial(
        pl.kernel,
        out_type=jax.ShapeDtypeStruct((NC, N, D), jnp.float32),
        mesh=_mesh,
        scratch_types=[
            pltpu.VMEM((RING, 2, CH), jnp.int32),
        ]
        + [pltpu.VMEM((CH, D), jnp.float32)] * NBUF
        + [pltpu.VMEM((ZB, D), jnp.float32),
           pltpu.VMEM_SHARED((N, D), jnp.float32)]
        + [pltpu.SemaphoreType.DMA] * NBUF
        + [pltpu.SemaphoreType.DMA] * NBUF
        + [pltpu.SemaphoreType.DMA] * RING,
    )
    def k(z_hbm, sd_hbm, out_hbm, idxr, *rest):
        r = rest[:NBUF]
        zbuf, acc = rest[NBUF], rest[NBUF + 1]
        sg = rest[NBUF + 2:2 * NBUF + 2]
        ss = rest[2 * NBUF + 2:3 * NBUF + 2]
        si = rest[3 * NBUF + 2:]
        cid = lax.axis_index("c")
        sid = lax.axis_index("s")
        wid = sid * NC + cid
        base_c = wid * N_CH

        _zero_fill(zbuf, ZB, D)

        def _init(start, size):
            @pl.loop(0, size // ZB)
            def _(i):
                pltpu.sync_copy(zbuf, acc.at[pl.ds(start + i * ZB, ZB)])

        _per_tile_rows(sid, _init)
        plsc.subcore_barrier()

        for off in range(RING):
            pltpu.async_copy(sd_hbm.at[base_c + off], idxr.at[off], si[off])
        for off in range(NBUF):
            pltpu.make_async_copy(sd_hbm.at[0], idxr.at[off], si[off]).wait()
            pltpu.async_copy(z_hbm.at[idxr.at[off, 0]], r[off], sg[off])

        def visit(q, off):
            b = off % NBUF
            nrow = (off + NBUF) % RING
            pltpu.make_async_copy(z_hbm.at[idxr.at[off, 0]], r[b], sg[b]).wait()
            pltpu.async_copy(r[b], acc.at[idxr.at[off, 1]], ss[b], add=True)
            pltpu.make_async_copy(r[b], acc.at[idxr.at[off, 1]], ss[b]).wait()

            def _prefetch_idx():
                pltpu.async_copy(sd_hbm.at[base_c + q + RING],
                                 idxr.at[off], si[off])

            def _next_gather():
                pltpu.make_async_copy(sd_hbm.at[0], idxr.at[nrow],
                                      si[nrow]).wait()
                pltpu.async_copy(z_hbm.at[idxr.at[nrow, 0]], r[b], sg[b])

            _maybe(q + RING < N_CH, _prefetch_idx)
            _maybe(q + NBUF < N_CH, _next_gather)

        @pl.loop(0, N_CH // RING)
        def _(i):
            for off in range(RING):
                visit(i * RING + off, off)

        for off in range(N_CH % RING):
            visit((N_CH // RING) * RING + off, off)

        plsc.subcore_barrier()

        def _drain(start, size):
            pltpu.sync_copy(
                acc.at[pl.ds(start, size)],
                out_hbm.at[cid, pl.ds(start, size)],
            )

        _per_tile_rows(sid, _drain)

    return k(z, sd)


def _dinv_block(degp):
    return lax.rsqrt(degp[0, :, 0:1] + degp[1, :, 0:1] + 1.0)


def _mm_body(x_ref, w_ref, o_ref):
    o_ref[...] = jnp.dot(
        x_ref[...], w_ref[...],
        preferred_element_type=jnp.float32, precision=lax.Precision.HIGHEST,
    )


def _tc_matmul(x, W):
    return pl.pallas_call(
        _mm_body,
        grid=(N // BM,),
        in_specs=[
            pl.BlockSpec((BM, D), lambda i: (i, 0)),
            pl.BlockSpec((D, D), lambda i: (0, 0)),
        ],
        out_specs=pl.BlockSpec((BM, D), lambda i: (i, 0)),
        out_shape=jax.ShapeDtypeStruct((N, D), jnp.float32),
    )(x, W)


def _z_body(degp_ref, y_ref, z_ref):
    z_ref[...] = _dinv_block(degp_ref) * y_ref[...]


def _tc_scale(degp, y):
    return pl.pallas_call(
        _z_body,
        grid=(N // BM,),
        in_specs=[
            pl.BlockSpec((NC, BM, 16), lambda i: (0, i, 0)),
            pl.BlockSpec((BM, D), lambda i: (i, 0)),
        ],
        out_specs=pl.BlockSpec((BM, D), lambda i: (i, 0)),
        out_shape=jax.ShapeDtypeStruct((N, D), jnp.float32),
    )(degp, y)


def _mid_body(degp_ref, p_ref, z1_ref, b1_ref, w2_ref, z2_ref):
    dinv = _dinv_block(degp_ref)
    agg = p_ref[0] + p_ref[1] + z1_ref[...]
    h = jnp.maximum(dinv * agg + b1_ref[...], 0.0)
    y2 = jnp.dot(
        h, w2_ref[...],
        preferred_element_type=jnp.float32, precision=lax.Precision.HIGHEST,
    )
    z2_ref[...] = dinv * y2


def _tc_mid(degp, p, z1, b1, W2):
    return pl.pallas_call(
        _mid_body,
        grid=(N // BM,),
        in_specs=[
            pl.BlockSpec((NC, BM, 16), lambda i: (0, i, 0)),
            pl.BlockSpec((NC, BM, D), lambda i: (0, i, 0)),
            pl.BlockSpec((BM, D), lambda i: (i, 0)),
            pl.BlockSpec((1, D), lambda i: (0, 0)),
            pl.BlockSpec((D, D), lambda i: (0, 0)),
        ],
        out_specs=pl.BlockSpec((BM, D), lambda i: (i, 0)),
        out_shape=jax.ShapeDtypeStruct((N, D), jnp.float32),
    )(degp, p, z1, b1, W2)


def _final_body(degp_ref, q_ref, z2_ref, b2_ref, o_ref):
    dinv = _dinv_block(degp_ref)
    g = dinv * (q_ref[0] + q_ref[1] + z2_ref[...]) + b2_ref[...]
    m = jnp.max(g, axis=1, keepdims=True)
    lse = m + jnp.log(jnp.sum(jnp.exp(g - m), axis=1, keepdims=True))
    o_ref[...] = g - lse


def _tc_final(degp, q, z2, b2):
    return pl.pallas_call(
        _final_body,
        grid=(N // BM,),
        in_specs=[
            pl.BlockSpec((NC, BM, 16), lambda i: (0, i, 0)),
            pl.BlockSpec((NC, BM, D), lambda i: (0, i, 0)),
            pl.BlockSpec((BM, D), lambda i: (i, 0)),
            pl.BlockSpec((1, D), lambda i: (0, 0)),
        ],
        out_specs=pl.BlockSpec((BM, D), lambda i: (i, 0)),
        out_shape=jax.ShapeDtypeStruct((N, D), jnp.float32),
    )(degp, q, z2, b2)


def kernel(x, edge_index, W1, b1, W2, b2):
    ei = edge_index.astype(jnp.int32)
    src = ei[0]
    dst = ei[1]
    sd = jnp.stack(
        [src.reshape(E // CH, CH), dst.reshape(E // CH, CH)], axis=1
    )
    b1 = b1.reshape(1, D)
    b2 = b2.reshape(1, D)

    degp = _sc_degree(dst)
    y1 = _tc_matmul(x, W1)
    z1 = _tc_scale(degp, y1)
    p = _sc_scatter(z1, sd)
    z2 = _tc_mid(degp, p, z1, b1, W2)
    q = _sc_scatter(z2, sd)
    return _tc_final(degp, q, z2, b2)
